# baseline (speedup 1.0000x reference)
import functools

import jax
import jax.numpy as jnp
from jax import lax
from jax.experimental import pallas as pl
from jax.experimental.pallas import tpu as pltpu
from jax.experimental.pallas import tpu_sc as plsc

_NC = 2
_NS = 16
_NW = _NC * _NS
_L = 16


@functools.lru_cache(maxsize=None)
def _make_gather(B: int, V: int, D: int, H: int):
    assert B % (_NW * _L) == 0 and D % 8 == 0
    bpw = B // _NW
    nchunk = bpw // _L
    rows_per_chunk = _L * H
    tb_per_w = bpw // 128

    mesh = plsc.VectorSubcoreMesh(core_axis_name="c", subcore_axis_name="s")

    @functools.partial(
        pl.kernel,
        out_type=jax.ShapeDtypeStruct((H, D // 8, B // 128, 8, 128),
                                      jnp.float32),
        mesh=mesh,
        scratch_types=[
            pltpu.VMEM((rows_per_chunk,), jnp.int32),
            pltpu.VMEM((rows_per_chunk, D), jnp.float32),
            pltpu.VMEM((H, D // 8, 8, _L), jnp.float32),
            pltpu.SemaphoreType.DMA,
            pltpu.SemaphoreType.DMA,
            pltpu.SemaphoreType.DMA,
        ],
        compiler_params=pltpu.CompilerParams(use_tc_tiling_on_sc=False,
                                             needs_layout_passes=False),
    )
    def gather(idx_hbm, table_hbm, out_hbm, idx_v, rows2, vout, isem, gsem,
               osem):
        wid = lax.axis_index("s") * _NC + lax.axis_index("c")
        b_base = wid * bpw
        lanes = lax.iota(jnp.int32, _L)
        lanesH = lanes * H
        group = 1
        while (group * H) % 8 != 0:
            group *= 2
        grows = group * H
        ngroups = _L // group

        def chunk_body(ci, carry):
            b0 = b_base + ci * _L
            pltpu.sync_copy(idx_hbm.at[pl.ds(b0 * H, rows_per_chunk)], idx_v)

            def add1(i, c):
                sl = pl.ds(i * _L, _L)
                idx_v[sl] = idx_v[sl] + 1
                return c

            lax.fori_loop(0, rows_per_chunk // _L, add1, 0)

            handles = [
                pltpu.async_copy(
                    table_hbm.at[idx_v.at[pl.ds(g * grows, grows)]],
                    rows2.at[pl.ds(g * grows, grows)], gsem)
                for g in range(ngroups)
            ]
            for hd in handles:
                hd.wait()

            def repack(h, c):
                rowvec = lanesH + h
                for d8 in range(D // 8):
                    for ds in range(8):
                        dvec = jnp.full((_L,), d8 * 8 + ds, jnp.int32)
                        v = plsc.load_gather(rows2, [rowvec, dvec])
                        vout[h, d8, ds, :] = v
                return c

            lax.fori_loop(0, H, repack, 0)

            tb = b0 // 128
            lb = b0 % 128
            ohandles = [
                pltpu.async_copy(
                    vout.at[h],
                    out_hbm.at[h, :, tb, :, pl.ds(lb, _L)], osem)
                for h in range(H)
            ]
            for hd in ohandles:
                hd.wait()
            return carry

        lax.fori_loop(0, nchunk, chunk_body, 0)

    return gather


def kernel(inputs, table):
    b, h = inputs.shape
    v, d = table.shape
    idx = inputs.reshape(-1)
    out5 = _make_gather(b, v, d, h)(idx, table)
    out = jnp.transpose(out5, (2, 4, 0, 1, 3)).reshape(b, h, d)
    return out


# speedup vs baseline: 1.0588x; 1.0588x over previous
import functools

import jax
import jax.numpy as jnp
from jax import lax
from jax.experimental import pallas as pl
from jax.experimental.pallas import tpu as pltpu
from jax.experimental.pallas import tpu_sc as plsc

_NC = 2
_NS = 16
_NW = _NC * _NS
_L = 16
_HG = 10
_TBW = 128


@functools.lru_cache(maxsize=None)
def _make_gather(B: int, V: int, D: int, H: int):
    assert B % (_NW * _TBW) == 0 and D % 8 == 0 and H % _HG == 0
    bpw = B // _NW
    ntb = bpw // _TBW
    npass = H // _HG
    nsub = _TBW // _L
    grows = _L * _HG

    mesh = plsc.VectorSubcoreMesh(core_axis_name="c", subcore_axis_name="s")

    @functools.partial(
        pl.kernel,
        out_type=jax.ShapeDtypeStruct((H, D // 8, B // _TBW, 8, _TBW),
                                      jnp.float32),
        mesh=mesh,
        scratch_types=[
            pltpu.VMEM((_TBW * H,), jnp.int32),
            pltpu.VMEM((grows,), jnp.int32),
            pltpu.VMEM((grows,), jnp.int32),
            pltpu.VMEM((grows, D), jnp.float32),
            pltpu.VMEM((grows, D), jnp.float32),
            pltpu.VMEM((_HG, D // 8, 8, _TBW), jnp.float32),
            pltpu.SemaphoreType.DMA,
            pltpu.SemaphoreType.DMA,
            pltpu.SemaphoreType.DMA,
            pltpu.SemaphoreType.DMA,
        ],
        compiler_params=pltpu.CompilerParams(use_tc_tiling_on_sc=False,
                                             needs_layout_passes=False),
    )
    def gather(idx_hbm, table_hbm, out_hbm, idx_v, didx0, didx1, rows0,
               rows1, vout, isem, gsem0, gsem1, osem):
        didx = (didx0, didx1)
        rows = (rows0, rows1)
        gsem = (gsem0, gsem1)
        wid = lax.axis_index("s") * _NC + lax.axis_index("c")
        lanes = lax.iota(jnp.int32, _L)
        lanesH = lanes * H

        def parity(s, fn):
            @pl.when(s % 2 == 0)
            def _():
                fn(0)

            @pl.when(s % 2 != 0)
            def _():
                fn(1)

        def tb_body(tb_i, carry):
            b0 = wid * bpw + tb_i * _TBW
            tb = b0 // _TBW
            pltpu.sync_copy(idx_hbm.at[pl.ds(b0 * H, _TBW * H)], idx_v)

            def add1(i, c):
                sl = pl.ds(i * _L, _L)
                idx_v[sl] = idx_v[sl] + 1
                return c

            lax.fori_loop(0, (_TBW * H) // _L, add1, 0)

            def gather_start(p, s, buf):
                svec = lanesH + (s * _L * H + p * _HG)

                def hsel(h, c3):
                    didx[buf][pl.ds(h * _L, _L)] = plsc.load_gather(
                        idx_v, [svec + h])
                    return c3

                lax.fori_loop(0, _HG, hsel, 0)
                pltpu.async_copy(table_hbm.at[didx[buf]], rows[buf],
                                 gsem[buf])

            def gather_wait(buf):
                pltpu.make_async_copy(table_hbm.at[didx[buf]], rows[buf],
                                      gsem[buf]).wait()

            def repack(s, buf):
                def rep_h(h, c3):
                    rowvec = lanes + h * _L
                    for d8 in range(D // 8):
                        for ds in range(8):
                            dvec = jnp.full((_L,), d8 * 8 + ds, jnp.int32)
                            v = plsc.load_gather(rows[buf], [rowvec, dvec])
                            vout[h, d8, ds, pl.ds(s * _L, _L)] = v
                    return c3

                lax.fori_loop(0, _HG, rep_h, 0)

            def pass_body(p, c):
                gather_start(p, 0, 0)

                def sub_body(s, c2):
                    parity(s + 1,
                           lambda b: pl.when(s + 1 < nsub)(
                               lambda: gather_start(p, s + 1, b)))
                    parity(s, gather_wait)
                    parity(s, lambda b: repack(s, b))
                    return c2

                lax.fori_loop(0, nsub, sub_body, 0)
                handles = [
                    pltpu.async_copy(vout.at[h],
                                     out_hbm.at[p * _HG + h, :, tb, :, :],
                                     osem)
                    for h in range(_HG)
                ]
                for hd in handles:
                    hd.wait()
                return c

            lax.fori_loop(0, npass, pass_body, 0)
            return carry

        lax.fori_loop(0, ntb, tb_body, 0)

    return gather


def kernel(inputs, table):
    b, h = inputs.shape
    v, d = table.shape
    idx = inputs.reshape(-1)
    out5 = _make_gather(b, v, d, h)(idx, table)
    out = jnp.transpose(out5, (2, 4, 0, 1, 3)).reshape(b, h, d)
    return out


# speedup vs baseline: 1.0606x; 1.0017x over previous
import functools

import jax
import jax.numpy as jnp
from jax import lax
from jax.experimental import pallas as pl
from jax.experimental.pallas import tpu as pltpu
from jax.experimental.pallas import tpu_sc as plsc

_NC = 2
_NS = 16
_NW = _NC * _NS
_L = 16
_HG = 10
_TBW = 128


@functools.lru_cache(maxsize=None)
def _make_gather(B: int, V: int, D: int, H: int):
    assert B % (_NW * _TBW) == 0 and D % 8 == 0 and H % _HG == 0
    bpw = B // _NW
    ntb = bpw // _TBW
    npass = H // _HG
    nsub = _TBW // _L
    grows = _L * _HG

    mesh = plsc.VectorSubcoreMesh(core_axis_name="c", subcore_axis_name="s")

    @functools.partial(
        pl.kernel,
        out_type=jax.ShapeDtypeStruct((H, D // 8, B // _TBW, 8, _TBW),
                                      jnp.float32),
        mesh=mesh,
        scratch_types=[
            pltpu.VMEM((_TBW * H,), jnp.int32),
            pltpu.VMEM((nsub * grows,), jnp.int32),
            pltpu.VMEM((nsub * grows, D), jnp.float32),
            pltpu.VMEM((_HG, D // 8, 8, _TBW), jnp.float32),
            pltpu.SemaphoreType.DMA,
            pltpu.SemaphoreType.DMA,
            pltpu.SemaphoreType.DMA,
        ],
        compiler_params=pltpu.CompilerParams(use_tc_tiling_on_sc=False,
                                             needs_layout_passes=False),
    )
    def gather(idx_hbm, table_hbm, out_hbm, idx_v, didx, rows, vout,
               isem, gsem, osem):
        wid = lax.axis_index("s") * _NC + lax.axis_index("c")
        lanes = lax.iota(jnp.int32, _L)
        lanesH = lanes * H

        def tb_body(tb_i, carry):
            b0 = wid * bpw + tb_i * _TBW
            tb = b0 // _TBW
            pltpu.sync_copy(idx_hbm.at[pl.ds(b0 * H, _TBW * H)], idx_v)

            def add1(i, c):
                sl = pl.ds(i * _L, _L)
                idx_v[sl] = idx_v[sl] + 1
                return c

            lax.fori_loop(0, (_TBW * H) // _L, add1, 0)

            def pass_body(p, c):
                def hsel(i, c3):
                    s = i // _HG
                    h = i % _HG
                    svec = lanesH + (s * _L * H + p * _HG + h)
                    didx[pl.ds(i * _L, _L)] = plsc.load_gather(idx_v, [svec])
                    return c3

                lax.fori_loop(0, nsub * _HG, hsel, 0)
                for s in range(nsub):
                    pltpu.async_copy(
                        table_hbm.at[didx.at[pl.ds(s * grows, grows)]],
                        rows.at[pl.ds(s * grows, grows)], gsem)

                for s in range(nsub):
                    pltpu.make_async_copy(
                        table_hbm.at[didx.at[pl.ds(s * grows, grows)]],
                        rows.at[pl.ds(s * grows, grows)], gsem).wait()

                    def rep_h(h, c3, s=s):
                        rowvec = lanes + (s * grows + h * _L)
                        for d8 in range(D // 8):
                            for ds in range(8):
                                dvec = jnp.full((_L,), d8 * 8 + ds, jnp.int32)
                                v = plsc.load_gather(rows, [rowvec, dvec])
                                vout[h, d8, ds, pl.ds(s * _L, _L)] = v
                        return c3

                    lax.fori_loop(0, _HG, rep_h, 0)

                handles = [
                    pltpu.async_copy(vout.at[h],
                                     out_hbm.at[p * _HG + h, :, tb, :, :],
                                     osem)
                    for h in range(_HG)
                ]
                for hd in handles:
                    hd.wait()
                return c

            lax.fori_loop(0, npass, pass_body, 0)
            return carry

        lax.fori_loop(0, ntb, tb_body, 0)

    return gather


def kernel(inputs, table):
    b, h = inputs.shape
    v, d = table.shape
    idx = inputs.reshape(-1)
    out5 = _make_gather(b, v, d, h)(idx, table)
    out = jnp.transpose(out5, (2, 4, 0, 1, 3)).reshape(b, h, d)
    return out


# speedup vs baseline: 1.2181x; 1.1485x over previous
import functools

import jax
import jax.numpy as jnp
from jax import lax
from jax.experimental import pallas as pl
from jax.experimental.pallas import tpu as pltpu
from jax.experimental.pallas import tpu_sc as plsc

_NC = 2
_NS = 16
_NW = _NC * _NS
_L = 16


@functools.lru_cache(maxsize=None)
def _make_gather(B: int, V: int, D: int, chunk: int, H: int):
    assert B % (_NW * chunk) == 0 and chunk % _L == 0 and chunk % H == 0
    bpw = B // _NW
    nchunk = bpw // chunk
    bchunk = chunk // H

    mesh = plsc.VectorSubcoreMesh(core_axis_name="c", subcore_axis_name="s")

    @functools.partial(
        pl.kernel,
        out_type=jax.ShapeDtypeStruct((B // H, H, D), jnp.float32),
        mesh=mesh,
        scratch_types=[
            pltpu.VMEM((chunk,), jnp.int32),
            pltpu.VMEM((chunk,), jnp.int32),
            pltpu.VMEM((chunk, D), jnp.float32),
            pltpu.VMEM((chunk, D), jnp.float32),
            pltpu.SemaphoreType.DMA,
            pltpu.SemaphoreType.DMA,
            pltpu.SemaphoreType.DMA,
            pltpu.SemaphoreType.DMA,
            pltpu.SemaphoreType.DMA,
            pltpu.SemaphoreType.DMA,
        ],
        compiler_params=pltpu.CompilerParams(use_tc_tiling_on_sc=False),
    )
    def gather(idx_hbm, table_hbm, out_hbm, idx0, idx1, rows0, rows1,
               isem0, isem1, gsem0, gsem1, osem0, osem1):
        idx_v = (idx0, idx1)
        rows_v = (rows0, rows1)
        isem = (isem0, isem1)
        gsem = (gsem0, gsem1)
        osem = (osem0, osem1)
        wid = lax.axis_index("s") * _NC + lax.axis_index("c")
        base = wid * bpw

        def idx_start(ci, p):
            off = base + ci * chunk
            return pltpu.async_copy(idx_hbm.at[pl.ds(off, chunk)], idx_v[p],
                                    isem[p])

        def out_start(ci, p):
            b0 = (base + ci * chunk) // H
            return [
                pltpu.async_copy(rows_v[p].at[pl.ds(k * H, H)],
                                 out_hbm.at[b0 + k], osem[p])
                for k in range(bchunk)
            ]

        pend_idx = idx_start(0, 0)
        pend_gather = [None, None]
        pend_out = [None, None]

        for ci in range(nchunk):
            p = ci % 2
            pend_idx.wait()

            def add1(i, c):
                sl = pl.ds(i * _L, _L)
                idx_v[p][sl] = idx_v[p][sl] + 1
                return c

            lax.fori_loop(0, chunk // _L, add1, 0)
            if pend_out[p] is not None:
                for h in pend_out[p]:
                    h.wait()
                pend_out[p] = None
            pend_gather[p] = pltpu.async_copy(table_hbm.at[idx_v[p]],
                                              rows_v[p], gsem[p])
            if ci + 1 < nchunk:
                if pend_gather[1 - p] is not None:
                    pend_gather[1 - p].wait()
                    pend_gather[1 - p] = None
                pend_idx = idx_start(ci + 1, 1 - p)
            if pend_gather[p] is not None:
                pend_gather[p].wait()
                pend_gather[p] = None
            pend_out[p] = out_start(ci, p)

        for p in range(2):
            if pend_out[p] is not None:
                for h in pend_out[p]:
                    h.wait()

    return gather


def kernel(inputs, table):
    b, h = inputs.shape
    v, d = table.shape
    idx = inputs.reshape(-1)
    return _make_gather(b * h, v, d, 1600, h)(idx, table)


# speedup vs baseline: 1.2297x; 1.0095x over previous
import functools

import jax
import jax.numpy as jnp
from jax import lax
from jax.experimental import pallas as pl
from jax.experimental.pallas import tpu as pltpu
from jax.experimental.pallas import tpu_sc as plsc

_NC = 2
_NS = 16
_NW = _NC * _NS
_L = 16


@functools.lru_cache(maxsize=None)
def _make_gather(B: int, V: int, D: int, chunk: int, H: int):
    assert B % (_NW * chunk) == 0 and chunk % _L == 0 and chunk % H == 0
    bpw = B // _NW
    nchunk = bpw // chunk
    bchunk = chunk // H

    mesh = plsc.VectorSubcoreMesh(core_axis_name="c", subcore_axis_name="s")

    @functools.partial(
        pl.kernel,
        out_type=jax.ShapeDtypeStruct((B // H, H, D), jnp.float32),
        mesh=mesh,
        scratch_types=[
            pltpu.VMEM((chunk,), jnp.int32),
            pltpu.VMEM((chunk,), jnp.int32),
            pltpu.VMEM((chunk, D), jnp.float32),
            pltpu.VMEM((chunk, D), jnp.float32),
            pltpu.SemaphoreType.DMA,
            pltpu.SemaphoreType.DMA,
            pltpu.SemaphoreType.DMA,
            pltpu.SemaphoreType.DMA,
            pltpu.SemaphoreType.DMA,
            pltpu.SemaphoreType.DMA,
        ],
        compiler_params=pltpu.CompilerParams(use_tc_tiling_on_sc=False),
    )
    def gather(idx_hbm, table_hbm, out_hbm, idx0, idx1, rows0, rows1,
               isem0, isem1, gsem0, gsem1, osem0, osem1):
        idx_v = (idx0, idx1)
        rows_v = (rows0, rows1)
        isem = (isem0, isem1)
        gsem = (gsem0, gsem1)
        osem = (osem0, osem1)
        wid = lax.axis_index("s") * _NC + lax.axis_index("c")
        base = wid * bpw

        def idx_start(ci, p):
            off = base + ci * chunk
            return pltpu.async_copy(idx_hbm.at[pl.ds(off, chunk)], idx_v[p],
                                    isem[p])

        def out_start(ci, p):
            b0 = (base + ci * chunk) // H
            return [
                pltpu.async_copy(rows_v[p].at[pl.ds(k * H, H)],
                                 out_hbm.at[b0 + k], osem[p])
                for k in range(bchunk)
            ]

        pend_idx = idx_start(0, 0)
        pend_gather = [None, None]
        pend_out = [None, None]

        for ci in range(nchunk):
            p = ci % 2
            pend_idx.wait()

            def add1(i, c):
                sl = pl.ds(i * _L, _L)
                idx_v[p][sl] = idx_v[p][sl] + 1
                return c

            lax.fori_loop(0, chunk // _L, add1, 0)
            if pend_out[p] is not None:
                for h in pend_out[p]:
                    h.wait()
                pend_out[p] = None
            pend_gather[p] = pltpu.async_copy(table_hbm.at[idx_v[p]],
                                              rows_v[p], gsem[p])
            if pend_gather[1 - p] is not None:
                pend_gather[1 - p].wait()
                pend_gather[1 - p] = None
                pend_out[1 - p] = out_start(ci - 1, 1 - p)
            if ci + 1 < nchunk:
                pend_idx = idx_start(ci + 1, 1 - p)

        pl_last = (nchunk - 1) % 2
        pend_gather[pl_last].wait()
        pend_out[pl_last] = out_start(nchunk - 1, pl_last)
        for p in range(2):
            if pend_out[p] is not None:
                for h in pend_out[p]:
                    h.wait()

    return gather


def kernel(inputs, table):
    b, h = inputs.shape
    v, d = table.shape
    idx = inputs.reshape(-1)
    return _make_gather(b * h, v, d, 1600, h)(idx, table)
